# baseline (speedup 1.0000x reference)
import functools

import jax
import jax.numpy as jnp
from jax import lax
from jax.experimental import pallas as pl
from jax.experimental.pallas import tpu as pltpu
from jax.experimental.pallas import tpu_sc as plsc

B = 1024
_CT = (((1,), (1,)), ((), ()))


def _tc1_body(xs_ref, idx_ref):
    x = xs_ref[:]
    sqh = 0.5 * jnp.sum(x * x, axis=1, keepdims=True)
    x_h = x.astype(jnp.bfloat16)
    x_m = (x - x_h.astype(jnp.float32)).astype(jnp.bfloat16)
    cat_l = jnp.concatenate([x_h, x_m, x_h], axis=1)
    cat_r = jnp.concatenate([x_h, x_h, x_m], axis=1)
    g = jax.lax.dot_general(cat_l, cat_r, (((1,), (1,)), ((), ())),
                            preferred_element_type=jnp.float32)
    score = jnp.transpose(sqh) - g
    row = jax.lax.broadcasted_iota(jnp.int32, (B, B), 0)
    col = jax.lax.broadcasted_iota(jnp.int32, (B, B), 1)
    score = jnp.where(row == col, jnp.float32(1e9), score)
    rowmin = jnp.min(score, axis=1, keepdims=True)
    cand = jnp.where(score == rowmin, col, B)
    idx_ref[:] = jnp.min(cand, axis=1, keepdims=True)


def _sc_gather(table, idx):
    info = plsc.get_sparse_core_info()
    nc, ns = info.num_cores, info.num_subcores
    nw = nc * ns
    b_per_w = B // nw
    d = table.shape[1]
    mesh = plsc.VectorSubcoreMesh(core_axis_name="c", subcore_axis_name="s")

    @functools.partial(
        pl.kernel, mesh=mesh,
        out_type=jax.ShapeDtypeStruct((B, d), jnp.float32),
        scratch_types=[
            pltpu.VMEM((b_per_w,), jnp.int32),
            pltpu.VMEM((b_per_w, d), jnp.float32),
            pltpu.SemaphoreType.DMA,
        ],
    )
    def k(table_hbm, idx_hbm, out_hbm, idx_v, rows_v, sem):
        wid = lax.axis_index("s") * nc + lax.axis_index("c")
        base = wid * b_per_w
        pltpu.sync_copy(idx_hbm.at[pl.ds(base, b_per_w)], idx_v)
        pltpu.async_copy(table_hbm.at[idx_v], rows_v, sem).wait()
        pltpu.sync_copy(rows_v, out_hbm.at[pl.ds(base, b_per_w)])

    return k(table, idx)


def _tc2_body(xs_ref, x0_ref, w1_ref, b1_ref, w2_ref, b2_ref, w3_ref, b3_ref,
              w4_ref, b4_ref, w5_ref, b5_ref, w6_ref, b6_ref,
              xhat_ref, zs_ref):
    x0 = x0_ref[:]
    a1 = jax.lax.dot_general(x0, w1_ref[:], _CT) + b1_ref[:][None, :]
    t1 = jax.lax.dot_general(xs_ref[:] - x0, w1_ref[:], _CT)
    h1 = jnp.maximum(a1, 0.0)
    dt1 = jnp.where(a1 > 0.0, t1, 0.0)

    a2 = jax.lax.dot_general(h1, w2_ref[:], _CT) + b2_ref[:][None, :]
    t2 = jax.lax.dot_general(dt1, w2_ref[:], _CT)
    h2 = jnp.maximum(a2, 0.0)
    dt2 = jnp.where(a2 > 0.0, t2, 0.0)

    z0 = jax.lax.dot_general(h2, w3_ref[:], _CT) + b3_ref[:][None, :]
    gz = jax.lax.dot_general(dt2, w3_ref[:], _CT)
    zs = z0 + gz
    zs_ref[:] = zs

    h4 = jnp.maximum(jax.lax.dot_general(zs, w4_ref[:], _CT) + b4_ref[:][None, :], 0.0)
    h5 = jnp.maximum(jax.lax.dot_general(h4, w5_ref[:], _CT) + b5_ref[:][None, :], 0.0)
    xhat_ref[:] = jax.lax.dot_general(h5, w6_ref[:], _CT) + b6_ref[:][None, :]


def kernel(xs, W1, b1, W2, b2, W3, b3, W4, b4, W5, b5, W6, b6):
    d = xs.shape[1]
    nz = W3.shape[0]
    tc1 = pl.pallas_call(
        _tc1_body,
        out_shape=jax.ShapeDtypeStruct((B, 1), jnp.int32),
    )
    idx = tc1(xs)
    x0 = _sc_gather(xs, jnp.reshape(idx, (B,)))
    tc2 = pl.pallas_call(
        _tc2_body,
        out_shape=(
            jax.ShapeDtypeStruct((B, d), jnp.float32),
            jax.ShapeDtypeStruct((B, nz), jnp.float32),
        ),
    )
    x_hats, zs = tc2(xs, x0, W1, b1, W2, b2, W3, b3, W4, b4, W5, b5, W6, b6)
    return (x_hats, zs)


# speedup vs baseline: 2.4807x; 2.4807x over previous
import jax
import jax.numpy as jnp
from jax.experimental import pallas as pl

B = 1024
_HI = jax.lax.Precision.HIGHEST
_CT = (((1,), (1,)), ((), ()))


def _body(xs_ref, w1_ref, b1_ref, w2_ref, b2_ref, w3_ref, b3_ref,
          w4_ref, b4_ref, w5_ref, b5_ref, w6_ref, b6_ref,
          xhat_ref, zs_ref):
    x = xs_ref[:]
    sqh = 0.5 * jnp.sum(x * x, axis=1, keepdims=True)
    x_h = x.astype(jnp.bfloat16)
    x_m = (x - x_h.astype(jnp.float32)).astype(jnp.bfloat16)
    cat_l = jnp.concatenate([x_h, x_m, x_h], axis=1)
    cat_r = jnp.concatenate([x_h, x_h, x_m], axis=1)
    g = jax.lax.dot_general(cat_l, cat_r, (((1,), (1,)), ((), ())),
                            preferred_element_type=jnp.float32)
    score = jnp.transpose(sqh) - g
    row = jax.lax.broadcasted_iota(jnp.int32, (B, B), 0)
    col = jax.lax.broadcasted_iota(jnp.int32, (B, B), 1)
    score = jnp.where(row == col, jnp.float32(1e9), score)
    rowmin = jnp.min(score, axis=1, keepdims=True)
    cand = jnp.where(score == rowmin, col, B)
    idx = jnp.min(cand, axis=1, keepdims=True)
    onehot = (cand == idx).astype(jnp.bfloat16)

    a1_all = jax.lax.dot_general(x, w1_ref[:], _CT)
    a_hi = a1_all.astype(jnp.bfloat16)
    r1 = a1_all - a_hi.astype(jnp.float32)
    a_mid = r1.astype(jnp.bfloat16)
    a_lo = (r1 - a_mid.astype(jnp.float32)).astype(jnp.bfloat16)
    limbs = jnp.concatenate([a_hi, a_mid, a_lo], axis=1)
    g3 = jax.lax.dot_general(onehot, limbs, (((1,), (0,)), ((), ())),
                             preferred_element_type=jnp.float32)
    a1_nn = (g3[:, 0:64] + g3[:, 64:128]) + g3[:, 128:192]
    a1 = a1_nn + b1_ref[:][None, :]
    t1 = a1_all - a1_nn
    h1 = jnp.maximum(a1, 0.0)
    dt1 = jnp.where(a1 > 0.0, t1, 0.0)

    a2 = jax.lax.dot_general(h1, w2_ref[:], _CT) + b2_ref[:][None, :]
    t2 = jax.lax.dot_general(dt1, w2_ref[:], _CT)
    h2 = jnp.maximum(a2, 0.0)
    dt2 = jnp.where(a2 > 0.0, t2, 0.0)

    z0 = jax.lax.dot_general(h2, w3_ref[:], _CT) + b3_ref[:][None, :]
    gz = jax.lax.dot_general(dt2, w3_ref[:], _CT)
    zs = z0 + gz
    zs_ref[:] = zs

    h4 = jnp.maximum(jax.lax.dot_general(zs, w4_ref[:], _CT) + b4_ref[:][None, :], 0.0)
    h5 = jnp.maximum(jax.lax.dot_general(h4, w5_ref[:], _CT) + b5_ref[:][None, :], 0.0)
    xhat_ref[:] = jax.lax.dot_general(h5, w6_ref[:], _CT) + b6_ref[:][None, :]


def kernel(xs, W1, b1, W2, b2, W3, b3, W4, b4, W5, b5, W6, b6):
    d = xs.shape[1]
    call = pl.pallas_call(
        _body,
        out_shape=(
            jax.ShapeDtypeStruct((B, d), jnp.float32),
            jax.ShapeDtypeStruct((B, W3.shape[0]), jnp.float32),
        ),
    )
    x_hats, zs = call(xs, W1, b1, W2, b2, W3, b3, W4, b4, W5, b5, W6, b6)
    return (x_hats, zs)
